# baseline (speedup 1.0000x reference)
import functools

import jax
import jax.numpy as jnp
from jax import lax
from jax.experimental import pallas as pl
from jax.experimental.pallas import tpu as pltpu
from jax.experimental.pallas import tpu_sc as plsc

_NC = 2
_NS = 16
_K = 128
_GB = 8


def _row_split(n):
    big = (-(-n // _NS) + 7) // 8 * 8
    last = n - (_NS - 1) * big
    assert last > 0 and last % 8 == 0
    return big, last


def _each_tile_rows(s, n, copy_fn):
    big, last = _row_split(n)

    @pl.when(s < _NS - 1)
    def _():
        copy_fn(s * big, big)

    @pl.when(s == _NS - 1)
    def _():
        copy_fn((_NS - 1) * big, last)


@functools.cache
def _make_agg(n, e, d):
    nw = _NC * _NS
    ew = e // nw
    nchunk = ew // _K
    mesh = plsc.VectorSubcoreMesh(core_axis_name="c", subcore_axis_name="s")

    ngroups = nchunk // _GB

    @functools.partial(
        pl.kernel,
        out_type=jax.ShapeDtypeStruct((_NC, n, d), jnp.float32),
        mesh=mesh,
        scratch_types=[
            pltpu.VMEM((2, _GB, _K), jnp.int32),
            pltpu.VMEM((2, _GB, _K), jnp.int32),
            pltpu.VMEM((2, _K, d), jnp.float32),
            pltpu.VMEM_SHARED((n, d), jnp.float32),
            pltpu.SemaphoreType.DMA,
            pltpu.SemaphoreType.DMA,
            pltpu.SemaphoreType.DMA,
            pltpu.SemaphoreType.DMA,
        ],
    )
    def agg(h_hbm, src_hbm, dst_hbm, z_hbm, out_hbm,
            srcg, dstg, rows, acc, isem0, isem1, gsem0, gsem1):
        c = lax.axis_index("c")
        s = lax.axis_index("s")
        wid = s * _NC + c
        _each_tile_rows(s, n, lambda st, sz: pltpu.sync_copy(
            z_hbm.at[pl.ds(st, sz)], acc.at[pl.ds(st, sz)]))
        plsc.subcore_barrier()
        row0 = wid * nchunk
        isems = (isem0, isem1)
        gsems = (gsem0, gsem1)

        def idx_start(gg, bg):
            pltpu.async_copy(src_hbm.at[pl.ds(row0 + gg * _GB, _GB)],
                             srcg.at[bg], isems[bg])
            pltpu.async_copy(dst_hbm.at[pl.ds(row0 + gg * _GB, _GB)],
                             dstg.at[bg], isems[bg])

        def idx_wait(bg):
            pltpu.make_async_copy(src_hbm.at[pl.ds(0, _GB)], srcg.at[bg],
                                  isems[bg]).wait()
            pltpu.make_async_copy(dst_hbm.at[pl.ds(0, _GB)], dstg.at[bg],
                                  isems[bg]).wait()

        idx_start(0, 0)
        if ngroups > 1:
            idx_start(1, 1)

        def group_body(gg, bg):
            idx_wait(bg)
            descs = [pltpu.async_copy(h_hbm.at[srcg.at[bg, 0]], rows.at[0],
                                      gsems[0])]
            for j in range(_GB):
                rb = j % 2
                if j + 1 < _GB:
                    descs.append(pltpu.async_copy(
                        h_hbm.at[srcg.at[bg, j + 1]], rows.at[1 - rb],
                        gsems[1 - rb]))
                descs[j].wait()
                pltpu.sync_copy(rows.at[rb], acc.at[dstg.at[bg, j]], add=True)
            @pl.when(gg + 2 < ngroups)
            def _():
                idx_start(gg + 2, bg)

        def group_pair(gp, carry):
            for bg in range(2):
                group_body(gp * 2 + bg, bg)
            return carry

        assert ngroups % 2 == 0
        lax.fori_loop(0, ngroups // 2, group_pair, 0)
        plsc.subcore_barrier()
        _each_tile_rows(s, n, lambda st, sz: pltpu.sync_copy(
            acc.at[pl.ds(st, sz)], out_hbm.at[c, pl.ds(st, sz)]))

    return agg


@functools.cache
def _make_deg(n, e):
    d = 16
    nw = _NC * _NS
    ew = e // nw
    nchunk = ew // _K
    mesh = plsc.VectorSubcoreMesh(core_axis_name="c", subcore_axis_name="s")

    ngroups = nchunk // _GB

    @functools.partial(
        pl.kernel,
        out_type=jax.ShapeDtypeStruct((_NC, n, d), jnp.float32),
        mesh=mesh,
        scratch_types=[
            pltpu.VMEM((2, _GB, _K), jnp.int32),
            pltpu.VMEM((_K, d), jnp.float32),
            pltpu.VMEM_SHARED((n, d), jnp.float32),
            pltpu.SemaphoreType.DMA,
            pltpu.SemaphoreType.DMA,
            pltpu.SemaphoreType.DMA,
        ],
    )
    def deg(dst_hbm, ones_hbm, z_hbm, out_hbm, dstg, ones, acc,
            isem0, isem1, ssem):
        c = lax.axis_index("c")
        s = lax.axis_index("s")
        wid = s * _NC + c
        pltpu.sync_copy(ones_hbm, ones)
        _each_tile_rows(s, n, lambda st, sz: pltpu.sync_copy(
            z_hbm.at[pl.ds(st, sz)], acc.at[pl.ds(st, sz)]))
        plsc.subcore_barrier()
        row0 = wid * nchunk
        isems = (isem0, isem1)

        def idx_start(gg, bg):
            pltpu.async_copy(dst_hbm.at[pl.ds(row0 + gg * _GB, _GB)],
                             dstg.at[bg], isems[bg])

        def idx_wait(bg):
            pltpu.make_async_copy(dst_hbm.at[pl.ds(0, _GB)], dstg.at[bg],
                                  isems[bg]).wait()

        idx_start(0, 0)
        if ngroups > 1:
            idx_start(1, 1)

        def group_body(gg, bg):
            idx_wait(bg)
            descs = [pltpu.async_copy(ones, acc.at[dstg.at[bg, j]], ssem,
                                      add=True) for j in range(_GB)]
            for dsc in descs:
                dsc.wait()

            @pl.when(gg + 2 < ngroups)
            def _():
                idx_start(gg + 2, bg)

        def group_pair(gp, carry):
            for bg in range(2):
                group_body(gp * 2 + bg, bg)
            return carry

        assert ngroups % 2 == 0
        lax.fori_loop(0, ngroups // 2, group_pair, 0)
        plsc.subcore_barrier()
        _each_tile_rows(s, n, lambda st, sz: pltpu.sync_copy(
            acc.at[pl.ds(st, sz)], out_hbm.at[c, pl.ds(st, sz)]))

    return deg


def _gelu(x):
    return 0.5 * x * (1.0 + lax.erf(x * 0.7071067811865476))


def _bn(x, g, b):
    m = jnp.mean(x, axis=0, keepdims=True)
    v = jnp.mean((x - m) * (x - m), axis=0, keepdims=True)
    return (x - m) * lax.rsqrt(v + 1e-5) * g + b


def _ln(x, g, b):
    m = jnp.mean(x, axis=-1, keepdims=True)
    v = jnp.mean((x - m) * (x - m), axis=-1, keepdims=True)
    return (x - m) * lax.rsqrt(v + 1e-5) * g + b


def _dot(a, b):
    return jnp.dot(a, b, preferred_element_type=jnp.float32)


def _ka_body(x_ref, win_ref, bin_ref, bng_ref, bnb_ref, wc0_ref, degp_ref,
             ha_ref, hb_ref, s_ref):
    nreal = x_ref.shape[0]
    h = _dot(x_ref[...], win_ref[...]) + bin_ref[...][None, :]
    h = _gelu(_bn(h, bng_ref[...][None, :], bnb_ref[...][None, :]))
    deg = 1.0 + degp_ref[0, :nreal, 0:1] + degp_ref[1, :nreal, 0:1]
    sv = lax.rsqrt(deg)
    s_ref[...] = sv
    hp = _dot(h, wc0_ref[...]) * sv
    ha_ref[...] = hp[:, :128]
    hb_ref[...] = hp[:, 128:]


def _tc_ka(x, w_in, b_in, bng, bnb, wc0, degp):
    n = x.shape[0]
    return pl.pallas_call(
        _ka_body,
        out_shape=[
            jax.ShapeDtypeStruct((n, 128), jnp.float32),
            jax.ShapeDtypeStruct((n, 128), jnp.float32),
            jax.ShapeDtypeStruct((n, 1), jnp.float32),
        ],
    )(x, w_in, b_in, bng, bnb, wc0, degp)


def _post(p0, p1, hself, sv, bc, bng, bnb, lng, lnb):
    agg = (p0 + p1 + hself) * sv + bc[None, :]
    t = _gelu(_bn(agg, bng[None, :], bnb[None, :]))
    return _ln(t, lng[None, :], lnb[None, :])


_B1 = 1000


def _kb1_pre(pa_ref, pb_ref, ha_ref, hb_ref, s_ref, bc_ref):
    sv = s_ref[...]
    ta = (pa_ref[0] + pa_ref[1] + ha_ref[...]) * sv + bc_ref[0, :128][None, :]
    tb = (pb_ref[0] + pb_ref[1] + hb_ref[...]) * sv + bc_ref[0, 128:][None, :]
    return ta, tb


def _kb1_stats_body(pa_ref, pb_ref, ha_ref, hb_ref, s_ref, bc_ref,
                    sum_ref, sq_ref):
    i = pl.program_id(0)
    ta, tb = _kb1_pre(pa_ref, pb_ref, ha_ref, hb_ref, s_ref, bc_ref)
    t = jnp.concatenate([ta, tb], axis=1)

    @pl.when(i == 0)
    def _():
        sum_ref[...] = jnp.zeros_like(sum_ref)
        sq_ref[...] = jnp.zeros_like(sq_ref)

    sum_ref[...] += jnp.sum(t, axis=0, keepdims=True)
    sq_ref[...] += jnp.sum(t * t, axis=0, keepdims=True)


def _kb1_apply_body(pa_ref, pb_ref, ha_ref, hb_ref, s_ref, bc_ref, bng_ref,
                    bnb_ref, lng_ref, lnb_ref, sum_ref, sq_ref, wc_ref,
                    out_ref, *, n):
    ta, tb = _kb1_pre(pa_ref, pb_ref, ha_ref, hb_ref, s_ref, bc_ref)
    m = sum_ref[...] / n
    v = sq_ref[...] / n - m * m
    r = lax.rsqrt(v + 1e-5)

    def bn_half(t, lo, hi):
        return ((t - m[0, lo:hi][None, :]) * r[0, lo:hi][None, :]
                * bng_ref[0, lo:hi][None, :] + bnb_ref[0, lo:hi][None, :])

    ga = _gelu(bn_half(ta, 0, 128))
    gb = _gelu(bn_half(tb, 128, 256))
    mr = (jnp.sum(ga, -1, keepdims=True) + jnp.sum(gb, -1, keepdims=True)) / 256.0
    da, db = ga - mr, gb - mr
    vr = (jnp.sum(da * da, -1, keepdims=True)
          + jnp.sum(db * db, -1, keepdims=True)) / 256.0
    rr = lax.rsqrt(vr + 1e-5)
    na = da * rr * lng_ref[0, :128][None, :] + lnb_ref[0, :128][None, :]
    nb = db * rr * lng_ref[0, 128:][None, :] + lnb_ref[0, 128:][None, :]
    out_ref[...] = (_dot(na, wc_ref[:128]) + _dot(nb, wc_ref[128:])) * s_ref[...]


def _tc_kb1(pa, pb, ha, hb, sv, bc, bng, bnb, lng, lnb, wc):
    n = sv.shape[0]
    nb_ = n // _B1
    bc2 = bc.reshape(1, 256)
    p_spec = pl.BlockSpec((2, _B1, 128), lambda i: (0, i, 0))
    h_spec = pl.BlockSpec((_B1, 128), lambda i: (i, 0))
    s_spec = pl.BlockSpec((_B1, 1), lambda i: (i, 0))
    full = pl.BlockSpec((1, 256), lambda i: (0, 0))
    stats = pl.pallas_call(
        _kb1_stats_body,
        grid=(nb_,),
        in_specs=[p_spec, p_spec, h_spec, h_spec, s_spec, full],
        out_specs=[full, full],
        out_shape=[jax.ShapeDtypeStruct((1, 256), jnp.float32)] * 2,
    )(pa, pb, ha, hb, sv, bc2)
    return pl.pallas_call(
        functools.partial(_kb1_apply_body, n=n),
        grid=(nb_,),
        in_specs=[p_spec, p_spec, h_spec, h_spec, s_spec, full, full, full,
                  full, full, full, full,
                  pl.BlockSpec((256, 128), lambda i: (0, 0))],
        out_specs=h_spec,
        out_shape=jax.ShapeDtypeStruct((n, 128), jnp.float32),
    )(pa, pb, ha, hb, sv, bc2, bng.reshape(1, 256), bnb.reshape(1, 256),
      lng.reshape(1, 256), lnb.reshape(1, 256), stats[0], stats[1], wc)


def _pad128(x):
    n, d = x.shape
    if d == 128:
        return x
    return jnp.concatenate([x, jnp.zeros((n, 128 - d), jnp.float32)], axis=1)


def _kb_body(p_ref, h_ref, s_ref, bc_ref, bng_ref, bnb_ref, lng_ref, lnb_ref,
             wc_ref, out_ref, *, din):
    nreal = h_ref.shape[0]
    sv = s_ref[...]
    xn = _post(p_ref[0, :nreal, :din], p_ref[1, :nreal, :din], h_ref[:, :din], sv,
               bc_ref[...], bng_ref[...], bnb_ref[...], lng_ref[...],
               lnb_ref[...])
    out_ref[...] = _pad128(_dot(xn, wc_ref[...]) * sv)


def _tc_kb(p, h, sv, bc, bng, bnb, lng, lnb, wc):
    n = h.shape[0]
    return pl.pallas_call(
        functools.partial(_kb_body, din=wc.shape[0]),
        out_shape=jax.ShapeDtypeStruct((n, 128), jnp.float32),
    )(p, h, sv, bc, bng, bnb, lng, lnb, wc)


def _kc_body(p_ref, h_ref, s_ref, bc_ref, bng_ref, bnb_ref, lng_ref, lnb_ref,
             wp1_ref, bp1_ref, lngp1_ref, lnbp1_ref,
             wp2_ref, bp2_ref, lngp2_ref, lnbp2_ref,
             wp3_ref, bp3_ref, wp4_ref, bp4_ref, out_ref, *, din):
    nreal = h_ref.shape[0]
    sv = s_ref[...]
    x5 = _post(p_ref[0, :nreal, :din], p_ref[1, :nreal, :din], h_ref[:, :din], sv,
               bc_ref[...], bng_ref[...], bnb_ref[...], lng_ref[...],
               lnb_ref[...])
    t = _dot(x5, wp1_ref[...]) + bp1_ref[...][None, :]
    t = _gelu(_ln(t, lngp1_ref[...][None, :], lnbp1_ref[...][None, :]))
    t = _dot(t, wp2_ref[...]) + bp2_ref[...][None, :]
    t = _gelu(_ln(t, lngp2_ref[...][None, :], lnbp2_ref[...][None, :]))
    t = _gelu(_dot(t, wp3_ref[...]) + bp3_ref[...][None, :])
    out_ref[...] = _dot(t, wp4_ref[...]) + bp4_ref[...][None, :]


def _tc_kc(p, h, sv, bc, bng, bnb, lng, lnb,
           wp1, bp1, lngp1, lnbp1, wp2, bp2, lngp2, lnbp2, wp3, bp3, wp4, bp4):
    n = h.shape[0]
    return pl.pallas_call(
        functools.partial(_kc_body, din=wp1.shape[0]),
        out_shape=jax.ShapeDtypeStruct((n, 1), jnp.float32),
    )(p, h, sv, bc, bng, bnb, lng, lnb,
      wp1, bp1, lngp1, lnbp1, wp2, bp2, lngp2, lnbp2, wp3, bp3, wp4, bp4)


def kernel(x, edge_index, W_in, b_in, bn_in_g, bn_in_b,
           Wc0, bc0, bng0, bnb0, lng0, lnb0,
           Wc1, bc1, bng1, bnb1, lng1, lnb1,
           Wc2, bc2, bng2, bnb2, lng2, lnb2,
           Wc3, bc3, bng3, bnb3, lng3, lnb3,
           Wc4, bc4, bng4, bnb4, lng4, lnb4,
           Wp1, bp1, lngp1, lnbp1, Wp2, bp2, lngp2, lnbp2,
           Wp3, bp3, Wp4, bp4):
    n = x.shape[0]
    e = edge_index.shape[1]
    nw = _NC * _NS
    ew = -(-e // (nw * _K * _GB * 2)) * (_K * _GB * 2)
    ep = nw * ew
    np_ = n + 8
    srcf = jnp.concatenate(
        [edge_index[0], jnp.zeros((ep - e,), jnp.int32)])
    dstf = jnp.concatenate(
        [edge_index[1], jnp.full((ep - e,), n, jnp.int32)])
    src = srcf.reshape(ep // _K, _K)
    dst = dstf.reshape(ep // _K, _K)
    ones16 = jnp.ones((_K, 16), jnp.float32)
    z16 = jnp.zeros((np_, 16), jnp.float32)

    degp = _make_deg(np_, ep)(dst, ones16, z16)
    ha, hb, sv = _tc_ka(x, W_in, b_in, bn_in_g, bn_in_b, Wc0, degp)

    def agg(h):
        d = h.shape[1]
        z = jnp.zeros((np_, d), jnp.float32)
        return _make_agg(np_, ep, d)(h, src, dst, z)

    h1 = _tc_kb1(agg(ha), agg(hb), ha, hb, sv, bc0, bng0, bnb0, lng0, lnb0, Wc1)
    h2 = _tc_kb(agg(h1), h1, sv, bc1, bng1, bnb1, lng1, lnb1, Wc2)
    h3 = _tc_kb(agg(h2), h2, sv, bc2, bng2, bnb2, lng2, lnb2, Wc3)
    h4 = _tc_kb(agg(h3), h3, sv, bc3, bng3, bnb3, lng3, lnb3, Wc4)
    out = _tc_kc(agg(h4), h4, sv, bc4, bng4, bnb4, lng4, lnb4,
                 Wp1, bp1, lngp1, lnbp1, Wp2, bp2, lngp2, lnbp2,
                 Wp3, bp3, Wp4, bp4)
    return out[:, 0]


# speedup vs baseline: 2.3645x; 2.3645x over previous
import functools

import jax
import jax.numpy as jnp
from jax import lax
from jax.experimental import pallas as pl
from jax.experimental.pallas import tpu as pltpu
from jax.experimental.pallas import tpu_sc as plsc

_NC = 2
_NS = 16
_K = 128
_GB = 8


def _row_split(n):
    big = (-(-n // _NS) + 7) // 8 * 8
    last = n - (_NS - 1) * big
    assert last > 0 and last % 8 == 0
    return big, last


def _each_tile_rows(s, n, copy_fn):
    big, last = _row_split(n)

    @pl.when(s < _NS - 1)
    def _():
        copy_fn(s * big, big)

    @pl.when(s == _NS - 1)
    def _():
        copy_fn((_NS - 1) * big, last)


@functools.cache
def _make_agg(n, e, d):
    nw = _NC * _NS
    ew = e // nw
    nchunk = ew // _K
    mesh = plsc.VectorSubcoreMesh(core_axis_name="c", subcore_axis_name="s")

    ngroups = nchunk // _GB

    @functools.partial(
        pl.kernel,
        out_type=jax.ShapeDtypeStruct((_NC, n, d), jnp.float32),
        mesh=mesh,
        scratch_types=[
            pltpu.VMEM((2, _GB, _K), jnp.int32),
            pltpu.VMEM((2, _GB, _K), jnp.int32),
            pltpu.VMEM((2, _K, d), jnp.float32),
            pltpu.VMEM_SHARED((n, d), jnp.float32),
            pltpu.SemaphoreType.DMA,
            pltpu.SemaphoreType.DMA,
            pltpu.SemaphoreType.DMA,
            pltpu.SemaphoreType.DMA,
        ],
    )
    def agg(h_hbm, src_hbm, dst_hbm, z_hbm, out_hbm,
            srcg, dstg, rows, acc, isem0, isem1, gsem0, gsem1):
        c = lax.axis_index("c")
        s = lax.axis_index("s")
        wid = s * _NC + c
        _each_tile_rows(s, n, lambda st, sz: pltpu.sync_copy(
            z_hbm.at[pl.ds(st, sz)], acc.at[pl.ds(st, sz)]))
        plsc.subcore_barrier()
        row0 = wid * nchunk
        isems = (isem0, isem1)
        gsems = (gsem0, gsem1)

        def idx_start(gg, bg):
            pltpu.async_copy(src_hbm.at[pl.ds(row0 + gg * _GB, _GB)],
                             srcg.at[bg], isems[bg])
            pltpu.async_copy(dst_hbm.at[pl.ds(row0 + gg * _GB, _GB)],
                             dstg.at[bg], isems[bg])

        def idx_wait(bg):
            pltpu.make_async_copy(src_hbm.at[pl.ds(0, _GB)], srcg.at[bg],
                                  isems[bg]).wait()
            pltpu.make_async_copy(dst_hbm.at[pl.ds(0, _GB)], dstg.at[bg],
                                  isems[bg]).wait()

        idx_start(0, 0)
        if ngroups > 1:
            idx_start(1, 1)

        def group_body(gg, bg):
            idx_wait(bg)
            for j in range(_GB):
                rb = j % 2
                pltpu.async_copy(h_hbm.at[srcg.at[bg, j]], rows.at[rb],
                                 gsems[rb]).wait()
                pltpu.sync_copy(rows.at[rb], acc.at[dstg.at[bg, j]], add=True)
            @pl.when(gg + 2 < ngroups)
            def _():
                idx_start(gg + 2, bg)

        def group_pair(gp, carry):
            for bg in range(2):
                group_body(gp * 2 + bg, bg)
            return carry

        assert ngroups % 2 == 0
        lax.fori_loop(0, ngroups // 2, group_pair, 0)
        plsc.subcore_barrier()
        _each_tile_rows(s, n, lambda st, sz: pltpu.sync_copy(
            acc.at[pl.ds(st, sz)], out_hbm.at[c, pl.ds(st, sz)]))

    return agg


@functools.cache
def _make_deg(n, e):
    d = 16
    nw = _NC * _NS
    ew = e // nw
    nchunk = ew // _K
    mesh = plsc.VectorSubcoreMesh(core_axis_name="c", subcore_axis_name="s")

    ngroups = nchunk // _GB

    @functools.partial(
        pl.kernel,
        out_type=jax.ShapeDtypeStruct((_NC, n, d), jnp.float32),
        mesh=mesh,
        scratch_types=[
            pltpu.VMEM((2, _GB, _K), jnp.int32),
            pltpu.VMEM((_K, d), jnp.float32),
            pltpu.VMEM_SHARED((n, d), jnp.float32),
            pltpu.SemaphoreType.DMA,
            pltpu.SemaphoreType.DMA,
            pltpu.SemaphoreType.DMA,
        ],
    )
    def deg(dst_hbm, ones_hbm, z_hbm, out_hbm, dstg, ones, acc,
            isem0, isem1, ssem):
        c = lax.axis_index("c")
        s = lax.axis_index("s")
        wid = s * _NC + c
        pltpu.sync_copy(ones_hbm, ones)
        _each_tile_rows(s, n, lambda st, sz: pltpu.sync_copy(
            z_hbm.at[pl.ds(st, sz)], acc.at[pl.ds(st, sz)]))
        plsc.subcore_barrier()
        row0 = wid * nchunk
        isems = (isem0, isem1)

        def idx_start(gg, bg):
            pltpu.async_copy(dst_hbm.at[pl.ds(row0 + gg * _GB, _GB)],
                             dstg.at[bg], isems[bg])

        def idx_wait(bg):
            pltpu.make_async_copy(dst_hbm.at[pl.ds(0, _GB)], dstg.at[bg],
                                  isems[bg]).wait()

        idx_start(0, 0)
        if ngroups > 1:
            idx_start(1, 1)

        def group_body(gg, bg):
            idx_wait(bg)
            for j in range(_GB):
                pltpu.sync_copy(ones, acc.at[dstg.at[bg, j]], add=True)

            @pl.when(gg + 2 < ngroups)
            def _():
                idx_start(gg + 2, bg)

        def group_pair(gp, carry):
            for bg in range(2):
                group_body(gp * 2 + bg, bg)
            return carry

        assert ngroups % 2 == 0
        lax.fori_loop(0, ngroups // 2, group_pair, 0)
        plsc.subcore_barrier()
        _each_tile_rows(s, n, lambda st, sz: pltpu.sync_copy(
            acc.at[pl.ds(st, sz)], out_hbm.at[c, pl.ds(st, sz)]))

    return deg


def _gelu(x):
    return 0.5 * x * (1.0 + lax.erf(x * 0.7071067811865476))


def _bn(x, g, b):
    m = jnp.mean(x, axis=0, keepdims=True)
    v = jnp.mean((x - m) * (x - m), axis=0, keepdims=True)
    return (x - m) * lax.rsqrt(v + 1e-5) * g + b


def _ln(x, g, b):
    m = jnp.mean(x, axis=-1, keepdims=True)
    v = jnp.mean((x - m) * (x - m), axis=-1, keepdims=True)
    return (x - m) * lax.rsqrt(v + 1e-5) * g + b


def _dot(a, b):
    return jnp.dot(a, b, preferred_element_type=jnp.float32)


def _ka_body(x_ref, win_ref, bin_ref, bng_ref, bnb_ref, wc0_ref, degp_ref,
             ha_ref, hb_ref, s_ref):
    nreal = x_ref.shape[0]
    h = _dot(x_ref[...], win_ref[...]) + bin_ref[...][None, :]
    h = _gelu(_bn(h, bng_ref[...][None, :], bnb_ref[...][None, :]))
    deg = 1.0 + degp_ref[0, :nreal, 0:1] + degp_ref[1, :nreal, 0:1]
    sv = lax.rsqrt(deg)
    s_ref[...] = sv
    hp = _dot(h, wc0_ref[...]) * sv
    ha_ref[...] = hp[:, :128]
    hb_ref[...] = hp[:, 128:]


def _tc_ka(x, w_in, b_in, bng, bnb, wc0, degp):
    n = x.shape[0]
    return pl.pallas_call(
        _ka_body,
        out_shape=[
            jax.ShapeDtypeStruct((n, 128), jnp.float32),
            jax.ShapeDtypeStruct((n, 128), jnp.float32),
            jax.ShapeDtypeStruct((n, 1), jnp.float32),
        ],
    )(x, w_in, b_in, bng, bnb, wc0, degp)


def _post(p0, p1, hself, sv, bc, bng, bnb, lng, lnb):
    agg = (p0 + p1 + hself) * sv + bc[None, :]
    t = _gelu(_bn(agg, bng[None, :], bnb[None, :]))
    return _ln(t, lng[None, :], lnb[None, :])


_B1 = 1000


def _kb1_pre(pa_ref, pb_ref, ha_ref, hb_ref, s_ref, bc_ref):
    sv = s_ref[...]
    ta = (pa_ref[0] + pa_ref[1] + ha_ref[...]) * sv + bc_ref[0, :128][None, :]
    tb = (pb_ref[0] + pb_ref[1] + hb_ref[...]) * sv + bc_ref[0, 128:][None, :]
    return ta, tb


def _kb1_stats_body(pa_ref, pb_ref, ha_ref, hb_ref, s_ref, bc_ref,
                    sum_ref, sq_ref):
    i = pl.program_id(0)
    ta, tb = _kb1_pre(pa_ref, pb_ref, ha_ref, hb_ref, s_ref, bc_ref)
    t = jnp.concatenate([ta, tb], axis=1)

    @pl.when(i == 0)
    def _():
        sum_ref[...] = jnp.zeros_like(sum_ref)
        sq_ref[...] = jnp.zeros_like(sq_ref)

    sum_ref[...] += jnp.sum(t, axis=0, keepdims=True)
    sq_ref[...] += jnp.sum(t * t, axis=0, keepdims=True)


def _kb1_apply_body(pa_ref, pb_ref, ha_ref, hb_ref, s_ref, bc_ref, bng_ref,
                    bnb_ref, lng_ref, lnb_ref, sum_ref, sq_ref, wc_ref,
                    out_ref, *, n):
    ta, tb = _kb1_pre(pa_ref, pb_ref, ha_ref, hb_ref, s_ref, bc_ref)
    m = sum_ref[...] / n
    v = sq_ref[...] / n - m * m
    r = lax.rsqrt(v + 1e-5)

    def bn_half(t, lo, hi):
        return ((t - m[0, lo:hi][None, :]) * r[0, lo:hi][None, :]
                * bng_ref[0, lo:hi][None, :] + bnb_ref[0, lo:hi][None, :])

    ga = _gelu(bn_half(ta, 0, 128))
    gb = _gelu(bn_half(tb, 128, 256))
    mr = (jnp.sum(ga, -1, keepdims=True) + jnp.sum(gb, -1, keepdims=True)) / 256.0
    da, db = ga - mr, gb - mr
    vr = (jnp.sum(da * da, -1, keepdims=True)
          + jnp.sum(db * db, -1, keepdims=True)) / 256.0
    rr = lax.rsqrt(vr + 1e-5)
    na = da * rr * lng_ref[0, :128][None, :] + lnb_ref[0, :128][None, :]
    nb = db * rr * lng_ref[0, 128:][None, :] + lnb_ref[0, 128:][None, :]
    out_ref[...] = (_dot(na, wc_ref[:128]) + _dot(nb, wc_ref[128:])) * s_ref[...]


def _tc_kb1(pa, pb, ha, hb, sv, bc, bng, bnb, lng, lnb, wc):
    n = sv.shape[0]
    nb_ = n // _B1
    bc2 = bc.reshape(1, 256)
    p_spec = pl.BlockSpec((2, _B1, 128), lambda i: (0, i, 0))
    h_spec = pl.BlockSpec((_B1, 128), lambda i: (i, 0))
    s_spec = pl.BlockSpec((_B1, 1), lambda i: (i, 0))
    full = pl.BlockSpec((1, 256), lambda i: (0, 0))
    stats = pl.pallas_call(
        _kb1_stats_body,
        grid=(nb_,),
        in_specs=[p_spec, p_spec, h_spec, h_spec, s_spec, full],
        out_specs=[full, full],
        out_shape=[jax.ShapeDtypeStruct((1, 256), jnp.float32)] * 2,
    )(pa, pb, ha, hb, sv, bc2)
    return pl.pallas_call(
        functools.partial(_kb1_apply_body, n=n),
        grid=(nb_,),
        in_specs=[p_spec, p_spec, h_spec, h_spec, s_spec, full, full, full,
                  full, full, full, full,
                  pl.BlockSpec((256, 128), lambda i: (0, 0))],
        out_specs=h_spec,
        out_shape=jax.ShapeDtypeStruct((n, 128), jnp.float32),
    )(pa, pb, ha, hb, sv, bc2, bng.reshape(1, 256), bnb.reshape(1, 256),
      lng.reshape(1, 256), lnb.reshape(1, 256), stats[0], stats[1], wc)


def _pad128(x):
    n, d = x.shape
    if d == 128:
        return x
    return jnp.concatenate([x, jnp.zeros((n, 128 - d), jnp.float32)], axis=1)


def _kb_body(p_ref, h_ref, s_ref, bc_ref, bng_ref, bnb_ref, lng_ref, lnb_ref,
             wc_ref, out_ref, *, din):
    nreal = h_ref.shape[0]
    sv = s_ref[...]
    xn = _post(p_ref[0, :nreal, :din], p_ref[1, :nreal, :din], h_ref[:, :din], sv,
               bc_ref[...], bng_ref[...], bnb_ref[...], lng_ref[...],
               lnb_ref[...])
    out_ref[...] = _pad128(_dot(xn, wc_ref[...]) * sv)


def _tc_kb(p, h, sv, bc, bng, bnb, lng, lnb, wc):
    n = h.shape[0]
    return pl.pallas_call(
        functools.partial(_kb_body, din=wc.shape[0]),
        out_shape=jax.ShapeDtypeStruct((n, 128), jnp.float32),
    )(p, h, sv, bc, bng, bnb, lng, lnb, wc)


def _kc_body(p_ref, h_ref, s_ref, bc_ref, bng_ref, bnb_ref, lng_ref, lnb_ref,
             wp1_ref, bp1_ref, lngp1_ref, lnbp1_ref,
             wp2_ref, bp2_ref, lngp2_ref, lnbp2_ref,
             wp3_ref, bp3_ref, wp4_ref, bp4_ref, out_ref, *, din):
    nreal = h_ref.shape[0]
    sv = s_ref[...]
    x5 = _post(p_ref[0, :nreal, :din], p_ref[1, :nreal, :din], h_ref[:, :din], sv,
               bc_ref[...], bng_ref[...], bnb_ref[...], lng_ref[...],
               lnb_ref[...])
    t = _dot(x5, wp1_ref[...]) + bp1_ref[...][None, :]
    t = _gelu(_ln(t, lngp1_ref[...][None, :], lnbp1_ref[...][None, :]))
    t = _dot(t, wp2_ref[...]) + bp2_ref[...][None, :]
    t = _gelu(_ln(t, lngp2_ref[...][None, :], lnbp2_ref[...][None, :]))
    t = _gelu(_dot(t, wp3_ref[...]) + bp3_ref[...][None, :])
    out_ref[...] = _dot(t, wp4_ref[...]) + bp4_ref[...][None, :]


def _tc_kc(p, h, sv, bc, bng, bnb, lng, lnb,
           wp1, bp1, lngp1, lnbp1, wp2, bp2, lngp2, lnbp2, wp3, bp3, wp4, bp4):
    n = h.shape[0]
    return pl.pallas_call(
        functools.partial(_kc_body, din=wp1.shape[0]),
        out_shape=jax.ShapeDtypeStruct((n, 1), jnp.float32),
    )(p, h, sv, bc, bng, bnb, lng, lnb,
      wp1, bp1, lngp1, lnbp1, wp2, bp2, lngp2, lnbp2, wp3, bp3, wp4, bp4)


def kernel(x, edge_index, W_in, b_in, bn_in_g, bn_in_b,
           Wc0, bc0, bng0, bnb0, lng0, lnb0,
           Wc1, bc1, bng1, bnb1, lng1, lnb1,
           Wc2, bc2, bng2, bnb2, lng2, lnb2,
           Wc3, bc3, bng3, bnb3, lng3, lnb3,
           Wc4, bc4, bng4, bnb4, lng4, lnb4,
           Wp1, bp1, lngp1, lnbp1, Wp2, bp2, lngp2, lnbp2,
           Wp3, bp3, Wp4, bp4):
    n = x.shape[0]
    e = edge_index.shape[1]
    nw = _NC * _NS
    ew = -(-e // (nw * _K * _GB * 2)) * (_K * _GB * 2)
    ep = nw * ew
    np_ = n + _K
    padw = ew - e // nw
    psrc = jnp.broadcast_to(jnp.arange(padw, dtype=jnp.int32), (nw, padw))
    pdst = jnp.broadcast_to(n + (jnp.arange(padw, dtype=jnp.int32) % _K),
                            (nw, padw))
    srcf = jnp.concatenate([edge_index[0].reshape(nw, e // nw), psrc], axis=1)
    dstf = jnp.concatenate([edge_index[1].reshape(nw, e // nw), pdst], axis=1)
    src = srcf.reshape(ep // _K, _K)
    dst = dstf.reshape(ep // _K, _K)
    ones16 = jnp.ones((_K, 16), jnp.float32)
    z16 = jnp.zeros((np_, 16), jnp.float32)

    degp = _make_deg(np_, ep)(dst, ones16, z16)
    ha, hb, sv = _tc_ka(x, W_in, b_in, bn_in_g, bn_in_b, Wc0, degp)

    def agg(h):
        d = h.shape[1]
        z = jnp.zeros((np_, d), jnp.float32)
        return _make_agg(np_, ep, d)(h, src, dst, z)

    h1 = _tc_kb1(agg(ha), agg(hb), ha, hb, sv, bc0, bng0, bnb0, lng0, lnb0, Wc1)
    h2 = _tc_kb(agg(h1), h1, sv, bc1, bng1, bnb1, lng1, lnb1, Wc2)
    h3 = _tc_kb(agg(h2), h2, sv, bc2, bng2, bnb2, lng2, lnb2, Wc3)
    h4 = _tc_kb(agg(h3), h3, sv, bc3, bng3, bnb3, lng3, lnb3, Wc4)
    out = _tc_kc(agg(h4), h4, sv, bc4, bng4, bnb4, lng4, lnb4,
                 Wp1, bp1, lngp1, lnbp1, Wp2, bp2, lngp2, lnbp2,
                 Wp3, bp3, Wp4, bp4)
    return out[:, 0]


# speedup vs baseline: 2.6536x; 1.1223x over previous
import functools

import jax
import jax.numpy as jnp
from jax import lax
from jax.experimental import pallas as pl
from jax.experimental.pallas import tpu as pltpu
from jax.experimental.pallas import tpu_sc as plsc

_NC = 2
_NS = 16
_K = 128
_GB = 8


def _row_split(n):
    big = (-(-n // _NS) + 7) // 8 * 8
    last = n - (_NS - 1) * big
    assert last > 0 and last % 8 == 0
    return big, last


def _each_tile_rows(s, n, copy_fn):
    big, last = _row_split(n)

    @pl.when(s < _NS - 1)
    def _():
        copy_fn(s * big, big)

    @pl.when(s == _NS - 1)
    def _():
        copy_fn((_NS - 1) * big, last)


@functools.cache
def _make_agg(n, e, d):
    nw = _NC * _NS
    ew = e // nw
    nchunk = ew // _K
    mesh = plsc.VectorSubcoreMesh(core_axis_name="c", subcore_axis_name="s")

    ngroups = nchunk // _GB

    @functools.partial(
        pl.kernel,
        out_type=jax.ShapeDtypeStruct((_NC, n, d), jnp.float32),
        mesh=mesh,
        scratch_types=[
            pltpu.VMEM((2, _GB, _K), jnp.int32),
            pltpu.VMEM((2, _GB, _K), jnp.int32),
            pltpu.VMEM((2, _K, d), jnp.float32),
            pltpu.VMEM_SHARED((n, d), jnp.float32),
            pltpu.SemaphoreType.DMA,
            pltpu.SemaphoreType.DMA,
            pltpu.SemaphoreType.DMA,
            pltpu.SemaphoreType.DMA,
        ],
    )
    def agg(h_hbm, src_hbm, dst_hbm, z_hbm, out_hbm,
            srcg, dstg, rows, acc, isem0, isem1, gsem0, gsem1):
        c = lax.axis_index("c")
        s = lax.axis_index("s")
        wid = s * _NC + c
        _each_tile_rows(s, n, lambda st, sz: pltpu.sync_copy(
            z_hbm.at[pl.ds(st, sz)], acc.at[pl.ds(st, sz)]))
        plsc.subcore_barrier()
        row0 = wid * nchunk
        isems = (isem0, isem1)
        gsems = (gsem0, gsem1)

        def idx_start(gg, bg):
            pltpu.async_copy(src_hbm.at[pl.ds(row0 + gg * _GB, _GB)],
                             srcg.at[bg], isems[bg])
            pltpu.async_copy(dst_hbm.at[pl.ds(row0 + gg * _GB, _GB)],
                             dstg.at[bg], isems[bg])

        def idx_wait(bg):
            pltpu.make_async_copy(src_hbm.at[pl.ds(0, _GB)], srcg.at[bg],
                                  isems[bg]).wait()
            pltpu.make_async_copy(dst_hbm.at[pl.ds(0, _GB)], dstg.at[bg],
                                  isems[bg]).wait()

        idx_start(0, 0)
        if ngroups > 1:
            idx_start(1, 1)

        def group_body(gg, bg):
            idx_wait(bg)
            for half in range(_GB // 2):
                descs = [pltpu.async_copy(
                    h_hbm.at[srcg.at[bg, half * 2 + i]], rows.at[i],
                    gsems[i]) for i in range(2)]
                for dsc in descs:
                    dsc.wait()
                for i in range(2):
                    pltpu.sync_copy(rows.at[i],
                                    acc.at[dstg.at[bg, half * 2 + i]],
                                    add=True)
            @pl.when(gg + 2 < ngroups)
            def _():
                idx_start(gg + 2, bg)

        def group_pair(gp, carry):
            for bg in range(2):
                group_body(gp * 2 + bg, bg)
            return carry

        assert ngroups % 2 == 0
        lax.fori_loop(0, ngroups // 2, group_pair, 0)
        plsc.subcore_barrier()
        _each_tile_rows(s, n, lambda st, sz: pltpu.sync_copy(
            acc.at[pl.ds(st, sz)], out_hbm.at[c, pl.ds(st, sz)]))

    return agg


@functools.cache
def _make_deg(n, e):
    d = 16
    nw = _NC * _NS
    ew = e // nw
    nchunk = ew // _K
    mesh = plsc.VectorSubcoreMesh(core_axis_name="c", subcore_axis_name="s")

    ngroups = nchunk // _GB

    @functools.partial(
        pl.kernel,
        out_type=jax.ShapeDtypeStruct((_NC, n, d), jnp.float32),
        mesh=mesh,
        scratch_types=[
            pltpu.VMEM((2, _GB, _K), jnp.int32),
            pltpu.VMEM((_K, d), jnp.float32),
            pltpu.VMEM_SHARED((n, d), jnp.float32),
            pltpu.SemaphoreType.DMA,
            pltpu.SemaphoreType.DMA,
            pltpu.SemaphoreType.DMA,
        ],
    )
    def deg(dst_hbm, ones_hbm, z_hbm, out_hbm, dstg, ones, acc,
            isem0, isem1, ssem):
        c = lax.axis_index("c")
        s = lax.axis_index("s")
        wid = s * _NC + c
        pltpu.sync_copy(ones_hbm, ones)
        _each_tile_rows(s, n, lambda st, sz: pltpu.sync_copy(
            z_hbm.at[pl.ds(st, sz)], acc.at[pl.ds(st, sz)]))
        plsc.subcore_barrier()
        row0 = wid * nchunk
        isems = (isem0, isem1)

        def idx_start(gg, bg):
            pltpu.async_copy(dst_hbm.at[pl.ds(row0 + gg * _GB, _GB)],
                             dstg.at[bg], isems[bg])

        def idx_wait(bg):
            pltpu.make_async_copy(dst_hbm.at[pl.ds(0, _GB)], dstg.at[bg],
                                  isems[bg]).wait()

        idx_start(0, 0)
        if ngroups > 1:
            idx_start(1, 1)

        def group_body(gg, bg):
            idx_wait(bg)
            for j in range(_GB):
                pltpu.sync_copy(ones, acc.at[dstg.at[bg, j]], add=True)

            @pl.when(gg + 2 < ngroups)
            def _():
                idx_start(gg + 2, bg)

        def group_pair(gp, carry):
            for bg in range(2):
                group_body(gp * 2 + bg, bg)
            return carry

        assert ngroups % 2 == 0
        lax.fori_loop(0, ngroups // 2, group_pair, 0)
        plsc.subcore_barrier()
        _each_tile_rows(s, n, lambda st, sz: pltpu.sync_copy(
            acc.at[pl.ds(st, sz)], out_hbm.at[c, pl.ds(st, sz)]))

    return deg


def _gelu(x):
    return 0.5 * x * (1.0 + lax.erf(x * 0.7071067811865476))


def _bn(x, g, b):
    m = jnp.mean(x, axis=0, keepdims=True)
    v = jnp.mean((x - m) * (x - m), axis=0, keepdims=True)
    return (x - m) * lax.rsqrt(v + 1e-5) * g + b


def _ln(x, g, b):
    m = jnp.mean(x, axis=-1, keepdims=True)
    v = jnp.mean((x - m) * (x - m), axis=-1, keepdims=True)
    return (x - m) * lax.rsqrt(v + 1e-5) * g + b


def _dot(a, b):
    return jnp.dot(a, b, preferred_element_type=jnp.float32)


def _ka_body(x_ref, win_ref, bin_ref, bng_ref, bnb_ref, wc0_ref, degp_ref,
             ha_ref, hb_ref, s_ref):
    nreal = x_ref.shape[0]
    h = _dot(x_ref[...], win_ref[...]) + bin_ref[...][None, :]
    h = _gelu(_bn(h, bng_ref[...][None, :], bnb_ref[...][None, :]))
    deg = 1.0 + degp_ref[0, :nreal, 0:1] + degp_ref[1, :nreal, 0:1]
    sv = lax.rsqrt(deg)
    s_ref[...] = sv
    hp = _dot(h, wc0_ref[...]) * sv
    ha_ref[...] = hp[:, :128]
    hb_ref[...] = hp[:, 128:]


def _tc_ka(x, w_in, b_in, bng, bnb, wc0, degp):
    n = x.shape[0]
    return pl.pallas_call(
        _ka_body,
        out_shape=[
            jax.ShapeDtypeStruct((n, 128), jnp.float32),
            jax.ShapeDtypeStruct((n, 128), jnp.float32),
            jax.ShapeDtypeStruct((n, 1), jnp.float32),
        ],
    )(x, w_in, b_in, bng, bnb, wc0, degp)


def _post(p0, p1, hself, sv, bc, bng, bnb, lng, lnb):
    agg = (p0 + p1 + hself) * sv + bc[None, :]
    t = _gelu(_bn(agg, bng[None, :], bnb[None, :]))
    return _ln(t, lng[None, :], lnb[None, :])


_B1 = 1000


def _kb1_pre(pa_ref, pb_ref, ha_ref, hb_ref, s_ref, bc_ref):
    sv = s_ref[...]
    ta = (pa_ref[0] + pa_ref[1] + ha_ref[...]) * sv + bc_ref[0, :128][None, :]
    tb = (pb_ref[0] + pb_ref[1] + hb_ref[...]) * sv + bc_ref[0, 128:][None, :]
    return ta, tb


def _kb1_stats_body(pa_ref, pb_ref, ha_ref, hb_ref, s_ref, bc_ref,
                    sum_ref, sq_ref):
    i = pl.program_id(0)
    ta, tb = _kb1_pre(pa_ref, pb_ref, ha_ref, hb_ref, s_ref, bc_ref)
    t = jnp.concatenate([ta, tb], axis=1)

    @pl.when(i == 0)
    def _():
        sum_ref[...] = jnp.zeros_like(sum_ref)
        sq_ref[...] = jnp.zeros_like(sq_ref)

    sum_ref[...] += jnp.sum(t, axis=0, keepdims=True)
    sq_ref[...] += jnp.sum(t * t, axis=0, keepdims=True)


def _kb1_apply_body(pa_ref, pb_ref, ha_ref, hb_ref, s_ref, bc_ref, bng_ref,
                    bnb_ref, lng_ref, lnb_ref, sum_ref, sq_ref, wc_ref,
                    out_ref, *, n):
    ta, tb = _kb1_pre(pa_ref, pb_ref, ha_ref, hb_ref, s_ref, bc_ref)
    m = sum_ref[...] / n
    v = sq_ref[...] / n - m * m
    r = lax.rsqrt(v + 1e-5)

    def bn_half(t, lo, hi):
        return ((t - m[0, lo:hi][None, :]) * r[0, lo:hi][None, :]
                * bng_ref[0, lo:hi][None, :] + bnb_ref[0, lo:hi][None, :])

    ga = _gelu(bn_half(ta, 0, 128))
    gb = _gelu(bn_half(tb, 128, 256))
    mr = (jnp.sum(ga, -1, keepdims=True) + jnp.sum(gb, -1, keepdims=True)) / 256.0
    da, db = ga - mr, gb - mr
    vr = (jnp.sum(da * da, -1, keepdims=True)
          + jnp.sum(db * db, -1, keepdims=True)) / 256.0
    rr = lax.rsqrt(vr + 1e-5)
    na = da * rr * lng_ref[0, :128][None, :] + lnb_ref[0, :128][None, :]
    nb = db * rr * lng_ref[0, 128:][None, :] + lnb_ref[0, 128:][None, :]
    out_ref[...] = (_dot(na, wc_ref[:128]) + _dot(nb, wc_ref[128:])) * s_ref[...]


def _tc_kb1(pa, pb, ha, hb, sv, bc, bng, bnb, lng, lnb, wc):
    n = sv.shape[0]
    nb_ = n // _B1
    bc2 = bc.reshape(1, 256)
    p_spec = pl.BlockSpec((2, _B1, 128), lambda i: (0, i, 0))
    h_spec = pl.BlockSpec((_B1, 128), lambda i: (i, 0))
    s_spec = pl.BlockSpec((_B1, 1), lambda i: (i, 0))
    full = pl.BlockSpec((1, 256), lambda i: (0, 0))
    stats = pl.pallas_call(
        _kb1_stats_body,
        grid=(nb_,),
        in_specs=[p_spec, p_spec, h_spec, h_spec, s_spec, full],
        out_specs=[full, full],
        out_shape=[jax.ShapeDtypeStruct((1, 256), jnp.float32)] * 2,
    )(pa, pb, ha, hb, sv, bc2)
    return pl.pallas_call(
        functools.partial(_kb1_apply_body, n=n),
        grid=(nb_,),
        in_specs=[p_spec, p_spec, h_spec, h_spec, s_spec, full, full, full,
                  full, full, full, full,
                  pl.BlockSpec((256, 128), lambda i: (0, 0))],
        out_specs=h_spec,
        out_shape=jax.ShapeDtypeStruct((n, 128), jnp.float32),
    )(pa, pb, ha, hb, sv, bc2, bng.reshape(1, 256), bnb.reshape(1, 256),
      lng.reshape(1, 256), lnb.reshape(1, 256), stats[0], stats[1], wc)


def _pad128(x):
    n, d = x.shape
    if d == 128:
        return x
    return jnp.concatenate([x, jnp.zeros((n, 128 - d), jnp.float32)], axis=1)


def _kb_body(p_ref, h_ref, s_ref, bc_ref, bng_ref, bnb_ref, lng_ref, lnb_ref,
             wc_ref, out_ref, *, din):
    nreal = h_ref.shape[0]
    sv = s_ref[...]
    xn = _post(p_ref[0, :nreal, :din], p_ref[1, :nreal, :din], h_ref[:, :din], sv,
               bc_ref[...], bng_ref[...], bnb_ref[...], lng_ref[...],
               lnb_ref[...])
    out_ref[...] = _pad128(_dot(xn, wc_ref[...]) * sv)


def _tc_kb(p, h, sv, bc, bng, bnb, lng, lnb, wc):
    n = h.shape[0]
    return pl.pallas_call(
        functools.partial(_kb_body, din=wc.shape[0]),
        out_shape=jax.ShapeDtypeStruct((n, 128), jnp.float32),
    )(p, h, sv, bc, bng, bnb, lng, lnb, wc)


def _kc_body(p_ref, h_ref, s_ref, bc_ref, bng_ref, bnb_ref, lng_ref, lnb_ref,
             wp1_ref, bp1_ref, lngp1_ref, lnbp1_ref,
             wp2_ref, bp2_ref, lngp2_ref, lnbp2_ref,
             wp3_ref, bp3_ref, wp4_ref, bp4_ref, out_ref, *, din):
    nreal = h_ref.shape[0]
    sv = s_ref[...]
    x5 = _post(p_ref[0, :nreal, :din], p_ref[1, :nreal, :din], h_ref[:, :din], sv,
               bc_ref[...], bng_ref[...], bnb_ref[...], lng_ref[...],
               lnb_ref[...])
    t = _dot(x5, wp1_ref[...]) + bp1_ref[...][None, :]
    t = _gelu(_ln(t, lngp1_ref[...][None, :], lnbp1_ref[...][None, :]))
    t = _dot(t, wp2_ref[...]) + bp2_ref[...][None, :]
    t = _gelu(_ln(t, lngp2_ref[...][None, :], lnbp2_ref[...][None, :]))
    t = _gelu(_dot(t, wp3_ref[...]) + bp3_ref[...][None, :])
    out_ref[...] = _dot(t, wp4_ref[...]) + bp4_ref[...][None, :]


def _tc_kc(p, h, sv, bc, bng, bnb, lng, lnb,
           wp1, bp1, lngp1, lnbp1, wp2, bp2, lngp2, lnbp2, wp3, bp3, wp4, bp4):
    n = h.shape[0]
    return pl.pallas_call(
        functools.partial(_kc_body, din=wp1.shape[0]),
        out_shape=jax.ShapeDtypeStruct((n, 1), jnp.float32),
    )(p, h, sv, bc, bng, bnb, lng, lnb,
      wp1, bp1, lngp1, lnbp1, wp2, bp2, lngp2, lnbp2, wp3, bp3, wp4, bp4)


def kernel(x, edge_index, W_in, b_in, bn_in_g, bn_in_b,
           Wc0, bc0, bng0, bnb0, lng0, lnb0,
           Wc1, bc1, bng1, bnb1, lng1, lnb1,
           Wc2, bc2, bng2, bnb2, lng2, lnb2,
           Wc3, bc3, bng3, bnb3, lng3, lnb3,
           Wc4, bc4, bng4, bnb4, lng4, lnb4,
           Wp1, bp1, lngp1, lnbp1, Wp2, bp2, lngp2, lnbp2,
           Wp3, bp3, Wp4, bp4):
    n = x.shape[0]
    e = edge_index.shape[1]
    nw = _NC * _NS
    ew = -(-e // (nw * _K * _GB * 2)) * (_K * _GB * 2)
    ep = nw * ew
    np_ = n + _K
    padw = ew - e // nw
    psrc = jnp.broadcast_to(jnp.arange(padw, dtype=jnp.int32), (nw, padw))
    pdst = jnp.broadcast_to(n + (jnp.arange(padw, dtype=jnp.int32) % _K),
                            (nw, padw))
    srcf = jnp.concatenate([edge_index[0].reshape(nw, e // nw), psrc], axis=1)
    dstf = jnp.concatenate([edge_index[1].reshape(nw, e // nw), pdst], axis=1)
    src = srcf.reshape(ep // _K, _K)
    dst = dstf.reshape(ep // _K, _K)
    ones16 = jnp.ones((_K, 16), jnp.float32)
    z16 = jnp.zeros((np_, 16), jnp.float32)

    degp = _make_deg(np_, ep)(dst, ones16, z16)
    ha, hb, sv = _tc_ka(x, W_in, b_in, bn_in_g, bn_in_b, Wc0, degp)

    def agg(h):
        d = h.shape[1]
        z = jnp.zeros((np_, d), jnp.float32)
        return _make_agg(np_, ep, d)(h, src, dst, z)

    h1 = _tc_kb1(agg(ha), agg(hb), ha, hb, sv, bc0, bng0, bnb0, lng0, lnb0, Wc1)
    h2 = _tc_kb(agg(h1), h1, sv, bc1, bng1, bnb1, lng1, lnb1, Wc2)
    h3 = _tc_kb(agg(h2), h2, sv, bc2, bng2, bnb2, lng2, lnb2, Wc3)
    h4 = _tc_kb(agg(h3), h3, sv, bc3, bng3, bnb3, lng3, lnb3, Wc4)
    out = _tc_kc(agg(h4), h4, sv, bc4, bng4, bnb4, lng4, lnb4,
                 Wp1, bp1, lngp1, lnbp1, Wp2, bp2, lngp2, lnbp2,
                 Wp3, bp3, Wp4, bp4)
    return out[:, 0]
